# baseline (speedup 1.0000x reference)
import functools

import jax
import jax.numpy as jnp
from jax import lax
from jax.experimental import pallas as pl
from jax.experimental.pallas import tpu as pltpu
from jax.experimental.pallas import tpu_sc as plsc

LVLS = 6
NBATCH = 16
NQ = 900
ND = 256
NCLS = 91
NFLAT = NQ * NCLS
NPAD = 81920
NVEC = NPAD // 16
KTOP = 100
OUTW = 112
HSTRIDE = 2048
NBINS1 = 2048
NBINS2 = 2048
NBINS3 = 1024
EQCAP = 64
CANDW = 192


def _heads_body(x_ref, r_ref, wc_ref, bc_ref, w1_ref, b1_ref, w2_ref, b2_ref,
                w3_ref, b3_ref, cls_ref, coord_ref, cls5_ref, coord5_ref):
    h = x_ref[0, 0]
    logits = jnp.dot(h, wc_ref[...]) + bc_ref[...]
    cls_ref[0] = logits[:, :NCLS]
    h1 = jnp.maximum(jnp.dot(h, w1_ref[...]) + b1_ref[...], 0.0)
    h2 = jnp.maximum(jnp.dot(h1, w2_ref[...]) + b2_ref[...], 0.0)
    t = jnp.dot(h2, w3_ref[...]) + b3_ref[...]
    r = jnp.clip(r_ref[0], 0.0, 1.0)
    inv = jnp.log(jnp.clip(r, 1e-5, None) / jnp.clip(1.0 - r, 1e-5, None))
    coord = jax.nn.sigmoid(t[:, :4] + inv)
    coord_ref[0] = coord
    cls5_ref[0] = logits[:, :NCLS]
    coord5_ref[0] = coord


def _run_heads(x4d, rr, wcp, bcp, w1, b1r, w2, b2r, w3p, b3p):
    n = LVLS * NBATCH
    full = lambda s: pl.BlockSpec(s, lambda i: (0,) * len(s))
    return pl.pallas_call(
        _heads_body,
        grid=(n,),
        in_specs=[
            pl.BlockSpec((1, 1, NQ, ND),
                         lambda i: (i // NBATCH, i % NBATCH, 0, 0)),
            pl.BlockSpec((1, NQ, 4), lambda i: (i, 0, 0)),
            full((ND, 128)), full((1, 128)),
            full((ND, ND)), full((1, ND)),
            full((ND, ND)), full((1, ND)),
            full((ND, 128)), full((1, 128)),
        ],
        out_specs=[
            pl.BlockSpec((1, NQ, NCLS), lambda i: (i, 0, 0)),
            pl.BlockSpec((1, NQ, 4), lambda i: (i, 0, 0)),
            pl.BlockSpec((1, NQ, NCLS), lambda i: (i % NBATCH, 0, 0)),
            pl.BlockSpec((1, NQ, 4), lambda i: (i % NBATCH, 0, 0)),
        ],
        out_shape=[
            jax.ShapeDtypeStruct((n, NQ, NCLS), jnp.float32),
            jax.ShapeDtypeStruct((n, NQ, 4), jnp.float32),
            jax.ShapeDtypeStruct((NBATCH, NQ, NCLS), jnp.float32),
            jax.ShapeDtypeStruct((NBATCH, NQ, 4), jnp.float32),
        ],
        compiler_params=pltpu.CompilerParams(
            dimension_semantics=("arbitrary",)),
    )(x4d, rr, wcp, bcp, w1, b1r, w2, b2r, w3p, b3p)


def _topk_body(bits_hbm, boxes_hbm, wv_hbm, hv_hbm, zz_hbm,
               scores_hbm, labels_hbm, boxout_hbm,
               keys_v, hist_v, tot_v, ck_v, ci_v, eqk_v, eqi_v,
               outs_u, outs_f, outl_v, rowb_v, box_v, wvec_v, hvec_v,
               outb_v):
    c = lax.axis_index("c")
    s = lax.axis_index("s")
    img = s * 2 + c

    @pl.when(img < NBATCH)
    def _work():
        iota = lax.iota(jnp.int32, 16)
        ones_i = jnp.ones((16,), jnp.int32)
        lane_base = iota * HSTRIDE

        pltpu.sync_copy(bits_hbm.at[img], keys_v)
        pltpu.sync_copy(boxes_hbm.at[img], box_v)
        pltpu.sync_copy(wv_hbm.at[img], wvec_v)
        pltpu.sync_copy(hv_hbm.at[img], hvec_v)

        def bcast_u32(x):
            return jnp.broadcast_to(x.astype(jnp.uint32), (16,))

        def bcast_i32(x):
            return jnp.broadcast_to(x.astype(jnp.int32), (16,))

        pltpu.sync_copy(zz_hbm, hist_v)

        @plsc.parallel_loop(0, NVEC, 1, unroll=8)
        def p1_body(i):
            raw = plsc.bitcast(keys_v[pl.ds(i * 16, 16)], jnp.uint32)
            sm = raw >> jnp.uint32(31)
            mm = (jnp.uint32(0) - sm) | jnp.uint32(0x80000000)
            key = raw ^ mm
            keys_v[pl.ds(i * 16, 16)] = plsc.bitcast(key, jnp.float32)
            b1 = (key >> jnp.uint32(21)).astype(jnp.int32)
            plsc.addupdate_scatter(hist_v, [lane_base + b1], ones_i)

        def _reduce_hist(nbins):
            def red_body(w, carry):
                acc = hist_v[pl.ds(w * 16, 16)]
                for l in range(1, 16):
                    acc = acc + hist_v[pl.ds(l * HSTRIDE + w * 16, 16)]
                tot_v[pl.ds(w * 16, 16)] = acc
                return carry
            lax.fori_loop(0, nbins // 16, red_body, 0)

        def _find_bin(nbins, kneed):
            def body(j, carry):
                above, fbin, cgt, hl = carry
                v = nbins // 16 - 1 - j
                h = tot_v[pl.ds(v * 16, 16)]
                suff = lax.rev(jnp.cumsum(lax.rev(h, (0,)), axis=0), (0,))
                tota = above + suff
                m = (tota >= kneed)
                cnt = jnp.sum(m.astype(jnp.int32))
                lane = cnt - 1
                sel = (iota == lane)
                tot_l = jnp.sum(jnp.where(sel, tota, 0))
                h_l = jnp.sum(jnp.where(sel, h, 0))
                hit = jnp.logical_and(cnt > 0, fbin < 0)
                fbin = jnp.where(hit, v * 16 + lane, fbin)
                cgt = jnp.where(hit, tot_l - h_l, cgt)
                hl = jnp.where(hit, h_l, hl)
                above = above + jnp.sum(h)
                return (above, fbin, cgt, hl)
            _, fbin, cgt, hl = lax.fori_loop(
                0, nbins // 16, body, (jnp.int32(0), jnp.int32(-1),
                                       jnp.int32(0), jnp.int32(0)))
            return fbin, cgt, hl

        _reduce_hist(NBINS1)
        b1f, cgt1, hl1 = _find_bin(NBINS1, jnp.int32(KTOP))
        kneed2 = jnp.int32(KTOP) - cgt1

        def _level2(_):
            pltpu.sync_copy(zz_hbm, hist_v)
            p1vec = bcast_u32(b1f)

            @plsc.parallel_loop(0, NVEC, 1, unroll=8)
            def p2_body(i):
                key = plsc.bitcast(keys_v[pl.ds(i * 16, 16)], jnp.uint32)
                m = (key >> jnp.uint32(21)) == p1vec
                b2 = ((key >> jnp.uint32(10))
                      & jnp.uint32(0x7FF)).astype(jnp.int32)
                plsc.addupdate_scatter(hist_v, [lane_base + b2], ones_i,
                                       mask=m)
            _reduce_hist(NBINS2)
            b2f, cgt2, hl2 = _find_bin(NBINS2, kneed2)
            kneed3 = kneed2 - cgt2
            pref22 = (b1f << 11) | b2f

            def _level3(_):
                pltpu.sync_copy(zz_hbm, hist_v)
                p22vec = bcast_u32(pref22)

                @plsc.parallel_loop(0, NVEC, 1, unroll=8)
                def p3_body(i):
                    key = plsc.bitcast(keys_v[pl.ds(i * 16, 16)], jnp.uint32)
                    m = (key >> jnp.uint32(10)) == p22vec
                    b3 = (key & jnp.uint32(0x3FF)).astype(jnp.int32)
                    plsc.addupdate_scatter(hist_v, [lane_base + b3], ones_i,
                                           mask=m)
                _reduce_hist(NBINS3)
                b3f, cgt3, _hl3 = _find_bin(NBINS3, kneed3)
                tthr = ((pref22.astype(jnp.uint32) << jnp.uint32(10))
                        | b3f.astype(jnp.uint32))
                return (tthr, tthr + jnp.uint32(1), kneed3 - cgt3)

            def _fast2(_):
                return ((pref22.astype(jnp.uint32) << jnp.uint32(10)),
                        ((pref22 + 1).astype(jnp.uint32) << jnp.uint32(10)),
                        hl2)

            return lax.cond(hl2 <= EQCAP, _fast2, _level3, 0)

        def _fast1(_):
            return ((b1f.astype(jnp.uint32) << jnp.uint32(21)),
                    ((b1f + 1).astype(jnp.uint32) << jnp.uint32(21)),
                    hl1)

        eq_lo, gt_lo, eq_cap = lax.cond(hl1 <= EQCAP, _fast1, _level2, 0)
        eq_lo_vec = jnp.broadcast_to(eq_lo, (16,))
        gt_lo_vec = jnp.broadcast_to(gt_lo, (16,))
        gt_ok = gt_lo_vec != jnp.uint32(0)

        zi = jnp.zeros((16,), jnp.int32)
        for v in range(CANDW // 16):
            ck_v[pl.ds(v * 16, 16)] = zi
            ci_v[pl.ds(v * 16, 16)] = zi
        for v in range(OUTW // 16):
            outs_u[pl.ds(v * 16, 16)] = zi
            outl_v[pl.ds(v * 16, 16)] = zi
            rowb_v[pl.ds(v * 16, 16)] = zi

        def col_body(i, carry):
            og, oe = carry
            key = plsc.bitcast(keys_v[pl.ds(i * 16, 16)], jnp.uint32)
            ge = key >= eq_lo_vec
            ng = jnp.sum(ge.astype(jnp.int32))

            def slow(og, oe):
                idxv = i * 16 + iota
                gt = jnp.logical_and(key >= gt_lo_vec, gt_ok)
                cg = jnp.sum(gt.astype(jnp.int32))
                ki = plsc.bitcast(key, jnp.int32)
                plsc.store_compressed(ck_v.at[pl.ds(og, 16)], ki, mask=gt)
                plsc.store_compressed(ci_v.at[pl.ds(og, 16)], idxv, mask=gt)
                eq = jnp.logical_and(ge, jnp.logical_not(gt))
                rank = jnp.cumsum(eq.astype(jnp.int32))
                keep = jnp.logical_and(eq, rank <= (eq_cap - oe))
                ce = jnp.sum(keep.astype(jnp.int32))
                plsc.store_compressed(eqk_v.at[pl.ds(oe, 16)], ki, mask=keep)
                plsc.store_compressed(eqi_v.at[pl.ds(oe, 16)], idxv, mask=keep)
                return (og + cg, oe + ce)

            return lax.cond(ng > 0, slow, lambda og, oe: (og, oe), og, oe)

        og, oe = lax.fori_loop(0, NVEC, col_body,
                               (jnp.int32(0), jnp.int32(0)))

        for j in range(7):
            m = (j * 16 + iota) < oe
            ek = eqk_v[pl.ds(j * 16, 16)]
            oldk = ck_v[pl.ds(og + j * 16, 16)]
            ck_v[pl.ds(og + j * 16, 16)] = jnp.where(m, ek, oldk)
            ev = eqi_v[pl.ds(j * 16, 16)]
            oldi = ci_v[pl.ds(og + j * 16, 16)]
            ci_v[pl.ds(og + j * 16, 16)] = jnp.where(m, ev, oldi)

        lane0 = iota == 0
        big = jnp.broadcast_to(jnp.int32(0x7FFFFFFF), (16,))
        nsel = CANDW // 16

        def sel_body(t, carry):
            ks = [plsc.bitcast(ck_v[pl.ds(v * 16, 16)], jnp.uint32)
                  for v in range(nsel)]
            vm = ks[0]
            for j in range(1, nsel):
                vm = jnp.maximum(vm, ks[j])
            g = jnp.max(vm)
            gvec = jnp.broadcast_to(g, (16,))
            im = big
            for j in range(nsel):
                ij = ci_v[pl.ds(j * 16, 16)]
                im = jnp.minimum(im, jnp.where(ks[j] == gvec, ij, big))
            gi = jnp.min(im)
            givec = bcast_i32(gi)
            tb = bcast_i32(t)
            plsc.store_scatter(outs_u, [tb], plsc.bitcast(gvec, jnp.int32),
                               mask=lane0)
            plsc.store_scatter(outl_v, [tb], givec % 91, mask=lane0)
            plsc.store_scatter(rowb_v, [tb], givec // 91, mask=lane0)
            for j in range(nsel):
                ij = ci_v[pl.ds(j * 16, 16)]
                hit = jnp.logical_and(ks[j] == gvec, ij == givec)
                ck_v[pl.ds(j * 16, 16)] = plsc.bitcast(
                    jnp.where(hit, jnp.uint32(0), ks[j]), jnp.int32)
            return carry

        lax.fori_loop(0, KTOP, sel_body, 0)

        for v in range(OUTW // 16):
            k = plsc.bitcast(outs_u[pl.ds(v * 16, 16)], jnp.uint32)
            posm = (k >> jnp.uint32(31)) == jnp.uint32(1)
            bits = jnp.where(posm, k & jnp.uint32(0x7FFFFFFF), ~k)
            x = plsc.bitcast(bits, jnp.float32)
            e = jnp.exp(-jnp.abs(x))
            outs_f[pl.ds(v * 16, 16)] = jnp.where(
                x >= 0, 1.0 / (1.0 + e), e / (1.0 + e))

        wv = wvec_v[...]
        hv = hvec_v[...]
        for v in range(OUTW // 16):
            r4 = rowb_v[pl.ds(v * 16, 16)] * 4
            cx = plsc.load_gather(box_v, [r4])
            cy = plsc.load_gather(box_v, [r4 + 1])
            w = plsc.load_gather(box_v, [r4 + 2])
            h = plsc.load_gather(box_v, [r4 + 3])
            pos4 = (v * 16 + iota) * 4
            plsc.store_scatter(outb_v, [pos4], (cx - 0.5 * w) * wv)
            plsc.store_scatter(outb_v, [pos4 + 1], (cy - 0.5 * h) * hv)
            plsc.store_scatter(outb_v, [pos4 + 2], (cx + 0.5 * w) * wv)
            plsc.store_scatter(outb_v, [pos4 + 3], (cy + 0.5 * h) * hv)

        pltpu.sync_copy(outs_f, scores_hbm.at[img])
        pltpu.sync_copy(outl_v, labels_hbm.at[img])
        pltpu.sync_copy(outb_v, boxout_hbm.at[img])


def _run_topk(bits, boxes5, wb, hb, zz):
    mesh = plsc.VectorSubcoreMesh(core_axis_name="c", subcore_axis_name="s",
                                  num_cores=2, num_subcores=16)
    fn = pl.kernel(
        _topk_body,
        out_type=[
            jax.ShapeDtypeStruct((NBATCH, OUTW), jnp.float32),
            jax.ShapeDtypeStruct((NBATCH, OUTW), jnp.int32),
            jax.ShapeDtypeStruct((NBATCH, OUTW * 4), jnp.float32),
        ],
        mesh=mesh,
        compiler_params=pltpu.CompilerParams(needs_layout_passes=False),
        scratch_types=[
            pltpu.VMEM((NPAD,), jnp.float32),
            pltpu.VMEM((16 * HSTRIDE,), jnp.int32),
            pltpu.VMEM((HSTRIDE,), jnp.int32),
            pltpu.VMEM((256,), jnp.int32),
            pltpu.VMEM((256,), jnp.int32),
            pltpu.VMEM((128,), jnp.int32),
            pltpu.VMEM((128,), jnp.int32),
            pltpu.VMEM((OUTW,), jnp.int32),
            pltpu.VMEM((OUTW,), jnp.float32),
            pltpu.VMEM((OUTW,), jnp.int32),
            pltpu.VMEM((OUTW,), jnp.int32),
            pltpu.VMEM((NQ * 4,), jnp.float32),
            pltpu.VMEM((16,), jnp.float32),
            pltpu.VMEM((16,), jnp.float32),
            pltpu.VMEM((OUTW * 4,), jnp.float32),
        ],
    )
    return fn(bits, boxes5, wb, hb, zz)


def kernel(x, init_reference, inter_references, orig_target_sizes,
           Wc, bc, W1, b1, W2, b2, W3, b3):
    wcp = jnp.pad(Wc, ((0, 0), (0, 128 - NCLS)))
    bcp = jnp.pad(bc, (0, 128 - NCLS)).reshape(1, 128)
    w3p = jnp.pad(W3, ((0, 0), (0, 124)))
    b3p = jnp.pad(b3, (0, 124)).reshape(1, 128)
    b1r = b1.reshape(1, ND)
    b2r = b2.reshape(1, ND)

    rr = jnp.concatenate([init_reference[None], inter_references[:LVLS - 1]],
                         axis=0).reshape(LVLS * NBATCH, NQ, 4)

    cls, coord, cls5, coord5 = _run_heads(x, rr, wcp, bcp, W1, b1r,
                                          W2, b2r, w3p, b3p)
    outputs_class = cls.reshape(LVLS, NBATCH, NQ, NCLS)
    outputs_coord = coord.reshape(LVLS, NBATCH, NQ, 4)

    logits5 = cls5.reshape(NBATCH, NFLAT)
    pad = jnp.full((NBATCH, NPAD - NFLAT), -jnp.inf, jnp.float32)
    bits = jnp.concatenate([logits5, pad], axis=1)
    boxes5 = coord5.reshape(NBATCH, NQ * 4)
    img_h = orig_target_sizes[:, 0].astype(jnp.float32)
    img_w = orig_target_sizes[:, 1].astype(jnp.float32)
    wb = jnp.broadcast_to(img_w[:, None], (NBATCH, 16)) * jnp.ones((NBATCH, 16))
    hb = jnp.broadcast_to(img_h[:, None], (NBATCH, 16)) * jnp.ones((NBATCH, 16))
    zz = jnp.zeros((16 * HSTRIDE,), jnp.int32)
    scores_p, labels_p, boxes_p = _run_topk(bits, boxes5, wb, hb, zz)
    boxes_p = boxes_p.reshape(NBATCH, OUTW, 4)
    return (outputs_class, outputs_coord,
            scores_p[:, :KTOP], labels_p[:, :KTOP], boxes_p[:, :KTOP, :])


# speedup vs baseline: 1.3280x; 1.3280x over previous
import functools

import jax
import jax.numpy as jnp
from jax import lax
from jax.experimental import pallas as pl
from jax.experimental.pallas import tpu as pltpu
from jax.experimental.pallas import tpu_sc as plsc

LVLS = 6
NBATCH = 16
NQ = 900
ND = 256
NCLS = 91
NFLAT = NQ * NCLS
NPAD = 81920
NVEC = NPAD // 16
KTOP = 100
OUTW = 112
HSTRIDE = 2048
NBINS1 = 2048
NBINS2 = 2048
NBINS3 = 1024
EQCAP = 64
CANDW = 192


def _heads_body(x_ref, r_ref, wc_ref, bc_ref, w1_ref, b1_ref, w2_ref, b2_ref,
                w3_ref, b3_ref, cls_ref, coord_ref):
    h = x_ref[0, 0]
    logits = jnp.dot(h, wc_ref[...]) + bc_ref[...]
    cls_ref[0] = logits[:, :NCLS]
    h1 = jnp.maximum(jnp.dot(h, w1_ref[...]) + b1_ref[...], 0.0)
    h2 = jnp.maximum(jnp.dot(h1, w2_ref[...]) + b2_ref[...], 0.0)
    t = jnp.dot(h2, w3_ref[...]) + b3_ref[...]
    r = jnp.clip(r_ref[0], 0.0, 1.0)
    inv = jnp.log(jnp.clip(r, 1e-5, None) / jnp.clip(1.0 - r, 1e-5, None))
    coord_ref[0] = jax.nn.sigmoid(t[:, :4] + inv)


def _heads_body4(x_ref, r_ref, wc_ref, bc_ref, w1_ref, b1_ref, w2_ref,
                 b2_ref, w3_ref, b3_ref, cls_ref, coord_ref):
    h = x_ref[0, 0]
    logits = jnp.dot(h, wc_ref[...]) + bc_ref[...]
    cls_ref[0, 0] = logits[:, :NCLS]
    h1 = jnp.maximum(jnp.dot(h, w1_ref[...]) + b1_ref[...], 0.0)
    h2 = jnp.maximum(jnp.dot(h1, w2_ref[...]) + b2_ref[...], 0.0)
    t = jnp.dot(h2, w3_ref[...]) + b3_ref[...]
    r = jnp.clip(r_ref[0], 0.0, 1.0)
    inv = jnp.log(jnp.clip(r, 1e-5, None) / jnp.clip(1.0 - r, 1e-5, None))
    coord_ref[0, 0] = jax.nn.sigmoid(t[:, :4] + inv)


def _run_heads(x4d, rr, wcp, bcp, w1, b1r, w2, b2r, w3p, b3p, lvl_lo, lvl_hi,
               full_out=False):
    nlvl = lvl_hi - lvl_lo
    n = nlvl * NBATCH
    full = lambda s: pl.BlockSpec(s, lambda i: (0,) * len(s))
    if full_out:
        out_specs = [
            pl.BlockSpec((1, 1, NQ, NCLS),
                         lambda i: (lvl_lo + i // NBATCH, i % NBATCH, 0, 0)),
            pl.BlockSpec((1, 1, NQ, 4),
                         lambda i: (lvl_lo + i // NBATCH, i % NBATCH, 0, 0)),
        ]
        out_shape = [
            jax.ShapeDtypeStruct((LVLS, NBATCH, NQ, NCLS), jnp.float32),
            jax.ShapeDtypeStruct((LVLS, NBATCH, NQ, 4), jnp.float32),
        ]
        body = _heads_body4
    else:
        out_specs = [
            pl.BlockSpec((1, NQ, NCLS), lambda i: (i, 0, 0)),
            pl.BlockSpec((1, NQ, 4), lambda i: (i, 0, 0)),
        ]
        out_shape = [
            jax.ShapeDtypeStruct((n, NQ, NCLS), jnp.float32),
            jax.ShapeDtypeStruct((n, NQ, 4), jnp.float32),
        ]
        body = _heads_body
    return pl.pallas_call(
        body,
        grid=(n,),
        in_specs=[
            pl.BlockSpec((1, 1, NQ, ND),
                         lambda i: (lvl_lo + i // NBATCH, i % NBATCH, 0, 0)),
            pl.BlockSpec((1, NQ, 4), lambda i: (lvl_lo * NBATCH + i, 0, 0)),
            full((ND, 128)), full((1, 128)),
            full((ND, ND)), full((1, ND)),
            full((ND, ND)), full((1, ND)),
            full((ND, 128)), full((1, 128)),
        ],
        out_specs=out_specs,
        out_shape=out_shape,
        compiler_params=pltpu.CompilerParams(
            dimension_semantics=("arbitrary",)),
    )(x4d, rr, wcp, bcp, w1, b1r, w2, b2r, w3p, b3p)


def _topk_body(bits_hbm, boxes_hbm, wv_hbm, hv_hbm, zz_hbm,
               scores_hbm, labels_hbm, boxout_hbm,
               keys_v, hist_v, tot_v, ck_v, ci_v, eqk_v, eqi_v,
               outs_u, outs_f, outl_v, rowb_v, box_v, wvec_v, hvec_v,
               outb_v):
    c = lax.axis_index("c")
    s = lax.axis_index("s")
    img = s * 2 + c

    @pl.when(img < NBATCH)
    def _work():
        iota = lax.iota(jnp.int32, 16)
        ones_i = jnp.ones((16,), jnp.int32)
        lane_base = iota * HSTRIDE

        pltpu.sync_copy(bits_hbm.at[img], keys_v)
        pltpu.sync_copy(boxes_hbm.at[img], box_v)
        pltpu.sync_copy(wv_hbm.at[img], wvec_v)
        pltpu.sync_copy(hv_hbm.at[img], hvec_v)

        def bcast_u32(x):
            return jnp.broadcast_to(x.astype(jnp.uint32), (16,))

        def bcast_i32(x):
            return jnp.broadcast_to(x.astype(jnp.int32), (16,))

        pltpu.sync_copy(zz_hbm, hist_v)

        @plsc.parallel_loop(0, NVEC, 1, unroll=8)
        def p1_body(i):
            raw = plsc.bitcast(keys_v[pl.ds(i * 16, 16)], jnp.uint32)
            sm = raw >> jnp.uint32(31)
            mm = (jnp.uint32(0) - sm) | jnp.uint32(0x80000000)
            key = raw ^ mm
            keys_v[pl.ds(i * 16, 16)] = plsc.bitcast(key, jnp.float32)
            b1 = (key >> jnp.uint32(21)).astype(jnp.int32)
            plsc.addupdate_scatter(hist_v, [lane_base + b1], ones_i)

        def _reduce_hist(nbins):
            def red_body(w, carry):
                acc = hist_v[pl.ds(w * 16, 16)]
                for l in range(1, 16):
                    acc = acc + hist_v[pl.ds(l * HSTRIDE + w * 16, 16)]
                tot_v[pl.ds(w * 16, 16)] = acc
                return carry
            lax.fori_loop(0, nbins // 16, red_body, 0)

        def _find_bin(nbins, kneed):
            def body(j, carry):
                above, fbin, cgt, hl = carry
                v = nbins // 16 - 1 - j
                h = tot_v[pl.ds(v * 16, 16)]
                suff = lax.rev(jnp.cumsum(lax.rev(h, (0,)), axis=0), (0,))
                tota = above + suff
                m = (tota >= kneed)
                cnt = jnp.sum(m.astype(jnp.int32))
                lane = cnt - 1
                sel = (iota == lane)
                tot_l = jnp.sum(jnp.where(sel, tota, 0))
                h_l = jnp.sum(jnp.where(sel, h, 0))
                hit = jnp.logical_and(cnt > 0, fbin < 0)
                fbin = jnp.where(hit, v * 16 + lane, fbin)
                cgt = jnp.where(hit, tot_l - h_l, cgt)
                hl = jnp.where(hit, h_l, hl)
                above = above + jnp.sum(h)
                return (above, fbin, cgt, hl)
            _, fbin, cgt, hl = lax.fori_loop(
                0, nbins // 16, body, (jnp.int32(0), jnp.int32(-1),
                                       jnp.int32(0), jnp.int32(0)))
            return fbin, cgt, hl

        _reduce_hist(NBINS1)
        b1f, cgt1, hl1 = _find_bin(NBINS1, jnp.int32(KTOP))
        kneed2 = jnp.int32(KTOP) - cgt1

        def _level2(_):
            pltpu.sync_copy(zz_hbm, hist_v)
            p1vec = bcast_u32(b1f)

            @plsc.parallel_loop(0, NVEC, 1, unroll=8)
            def p2_body(i):
                key = plsc.bitcast(keys_v[pl.ds(i * 16, 16)], jnp.uint32)
                m = (key >> jnp.uint32(21)) == p1vec
                b2 = ((key >> jnp.uint32(10))
                      & jnp.uint32(0x7FF)).astype(jnp.int32)
                plsc.addupdate_scatter(hist_v, [lane_base + b2], ones_i,
                                       mask=m)
            _reduce_hist(NBINS2)
            b2f, cgt2, hl2 = _find_bin(NBINS2, kneed2)
            kneed3 = kneed2 - cgt2
            pref22 = (b1f << 11) | b2f

            def _level3(_):
                pltpu.sync_copy(zz_hbm, hist_v)
                p22vec = bcast_u32(pref22)

                @plsc.parallel_loop(0, NVEC, 1, unroll=8)
                def p3_body(i):
                    key = plsc.bitcast(keys_v[pl.ds(i * 16, 16)], jnp.uint32)
                    m = (key >> jnp.uint32(10)) == p22vec
                    b3 = (key & jnp.uint32(0x3FF)).astype(jnp.int32)
                    plsc.addupdate_scatter(hist_v, [lane_base + b3], ones_i,
                                           mask=m)
                _reduce_hist(NBINS3)
                b3f, cgt3, _hl3 = _find_bin(NBINS3, kneed3)
                tthr = ((pref22.astype(jnp.uint32) << jnp.uint32(10))
                        | b3f.astype(jnp.uint32))
                return (tthr, tthr + jnp.uint32(1), kneed3 - cgt3)

            def _fast2(_):
                return ((pref22.astype(jnp.uint32) << jnp.uint32(10)),
                        ((pref22 + 1).astype(jnp.uint32) << jnp.uint32(10)),
                        hl2)

            return lax.cond(hl2 <= EQCAP, _fast2, _level3, 0)

        def _fast1(_):
            return ((b1f.astype(jnp.uint32) << jnp.uint32(21)),
                    ((b1f + 1).astype(jnp.uint32) << jnp.uint32(21)),
                    hl1)

        eq_lo, gt_lo, eq_cap = lax.cond(hl1 <= EQCAP, _fast1, _level2, 0)
        eq_lo_vec = jnp.broadcast_to(eq_lo, (16,))
        gt_lo_vec = jnp.broadcast_to(gt_lo, (16,))
        gt_ok = gt_lo_vec != jnp.uint32(0)

        zi = jnp.zeros((16,), jnp.int32)
        for v in range(CANDW // 16):
            ck_v[pl.ds(v * 16, 16)] = zi
            ci_v[pl.ds(v * 16, 16)] = zi
        for v in range(OUTW // 16):
            outs_u[pl.ds(v * 16, 16)] = zi
            outl_v[pl.ds(v * 16, 16)] = zi
            rowb_v[pl.ds(v * 16, 16)] = zi

        def col_body(i, carry):
            og, oe = carry
            key = plsc.bitcast(keys_v[pl.ds(i * 16, 16)], jnp.uint32)
            ge = key >= eq_lo_vec
            ng = jnp.sum(ge.astype(jnp.int32))

            def slow(og, oe):
                idxv = i * 16 + iota
                gt = jnp.logical_and(key >= gt_lo_vec, gt_ok)
                cg = jnp.sum(gt.astype(jnp.int32))
                ki = plsc.bitcast(key, jnp.int32)
                plsc.store_compressed(ck_v.at[pl.ds(og, 16)], ki, mask=gt)
                plsc.store_compressed(ci_v.at[pl.ds(og, 16)], idxv, mask=gt)
                eq = jnp.logical_and(ge, jnp.logical_not(gt))
                rank = jnp.cumsum(eq.astype(jnp.int32))
                keep = jnp.logical_and(eq, rank <= (eq_cap - oe))
                ce = jnp.sum(keep.astype(jnp.int32))
                plsc.store_compressed(eqk_v.at[pl.ds(oe, 16)], ki, mask=keep)
                plsc.store_compressed(eqi_v.at[pl.ds(oe, 16)], idxv, mask=keep)
                return (og + cg, oe + ce)

            return lax.cond(ng > 0, slow, lambda og, oe: (og, oe), og, oe)

        og, oe = lax.fori_loop(0, NVEC, col_body,
                               (jnp.int32(0), jnp.int32(0)))

        for j in range(7):
            m = (j * 16 + iota) < oe
            ek = eqk_v[pl.ds(j * 16, 16)]
            oldk = ck_v[pl.ds(og + j * 16, 16)]
            ck_v[pl.ds(og + j * 16, 16)] = jnp.where(m, ek, oldk)
            ev = eqi_v[pl.ds(j * 16, 16)]
            oldi = ci_v[pl.ds(og + j * 16, 16)]
            ci_v[pl.ds(og + j * 16, 16)] = jnp.where(m, ev, oldi)

        lane0 = iota == 0
        big = jnp.broadcast_to(jnp.int32(0x7FFFFFFF), (16,))
        nsel = CANDW // 16

        def sel_body(t, carry):
            ks = [plsc.bitcast(ck_v[pl.ds(v * 16, 16)], jnp.uint32)
                  for v in range(nsel)]
            vm = ks[0]
            for j in range(1, nsel):
                vm = jnp.maximum(vm, ks[j])
            g = jnp.max(vm)
            gvec = jnp.broadcast_to(g, (16,))
            im = big
            for j in range(nsel):
                ij = ci_v[pl.ds(j * 16, 16)]
                im = jnp.minimum(im, jnp.where(ks[j] == gvec, ij, big))
            gi = jnp.min(im)
            givec = bcast_i32(gi)
            tb = bcast_i32(t)
            plsc.store_scatter(outs_u, [tb], plsc.bitcast(gvec, jnp.int32),
                               mask=lane0)
            plsc.store_scatter(outl_v, [tb], givec % 91, mask=lane0)
            plsc.store_scatter(rowb_v, [tb], givec // 91, mask=lane0)
            for j in range(nsel):
                ij = ci_v[pl.ds(j * 16, 16)]
                hit = jnp.logical_and(ks[j] == gvec, ij == givec)
                ck_v[pl.ds(j * 16, 16)] = plsc.bitcast(
                    jnp.where(hit, jnp.uint32(0), ks[j]), jnp.int32)
            return carry

        lax.fori_loop(0, KTOP, sel_body, 0)

        for v in range(OUTW // 16):
            k = plsc.bitcast(outs_u[pl.ds(v * 16, 16)], jnp.uint32)
            posm = (k >> jnp.uint32(31)) == jnp.uint32(1)
            bits = jnp.where(posm, k & jnp.uint32(0x7FFFFFFF), ~k)
            x = plsc.bitcast(bits, jnp.float32)
            e = jnp.exp(-jnp.abs(x))
            outs_f[pl.ds(v * 16, 16)] = jnp.where(
                x >= 0, 1.0 / (1.0 + e), e / (1.0 + e))

        wv = wvec_v[...]
        hv = hvec_v[...]
        for v in range(OUTW // 16):
            r4 = rowb_v[pl.ds(v * 16, 16)] * 4
            cx = plsc.load_gather(box_v, [r4])
            cy = plsc.load_gather(box_v, [r4 + 1])
            w = plsc.load_gather(box_v, [r4 + 2])
            h = plsc.load_gather(box_v, [r4 + 3])
            pos4 = (v * 16 + iota) * 4
            plsc.store_scatter(outb_v, [pos4], (cx - 0.5 * w) * wv)
            plsc.store_scatter(outb_v, [pos4 + 1], (cy - 0.5 * h) * hv)
            plsc.store_scatter(outb_v, [pos4 + 2], (cx + 0.5 * w) * wv)
            plsc.store_scatter(outb_v, [pos4 + 3], (cy + 0.5 * h) * hv)

        pltpu.sync_copy(outs_f, scores_hbm.at[img])
        pltpu.sync_copy(outl_v, labels_hbm.at[img])
        pltpu.sync_copy(outb_v, boxout_hbm.at[img])


def _run_topk(bits, boxes5, wb, hb, zz):
    mesh = plsc.VectorSubcoreMesh(core_axis_name="c", subcore_axis_name="s",
                                  num_cores=2, num_subcores=16)
    fn = pl.kernel(
        _topk_body,
        out_type=[
            jax.ShapeDtypeStruct((NBATCH, OUTW), jnp.float32),
            jax.ShapeDtypeStruct((NBATCH, OUTW), jnp.int32),
            jax.ShapeDtypeStruct((NBATCH, OUTW * 4), jnp.float32),
        ],
        mesh=mesh,
        compiler_params=pltpu.CompilerParams(needs_layout_passes=False),
        scratch_types=[
            pltpu.VMEM((NPAD,), jnp.float32),
            pltpu.VMEM((16 * HSTRIDE,), jnp.int32),
            pltpu.VMEM((HSTRIDE,), jnp.int32),
            pltpu.VMEM((256,), jnp.int32),
            pltpu.VMEM((256,), jnp.int32),
            pltpu.VMEM((128,), jnp.int32),
            pltpu.VMEM((128,), jnp.int32),
            pltpu.VMEM((OUTW,), jnp.int32),
            pltpu.VMEM((OUTW,), jnp.float32),
            pltpu.VMEM((OUTW,), jnp.int32),
            pltpu.VMEM((OUTW,), jnp.int32),
            pltpu.VMEM((NQ * 4,), jnp.float32),
            pltpu.VMEM((16,), jnp.float32),
            pltpu.VMEM((16,), jnp.float32),
            pltpu.VMEM((OUTW * 4,), jnp.float32),
        ],
    )
    return fn(bits, boxes5, wb, hb, zz)


def kernel(x, init_reference, inter_references, orig_target_sizes,
           Wc, bc, W1, b1, W2, b2, W3, b3):
    wcp = jnp.pad(Wc, ((0, 0), (0, 128 - NCLS)))
    bcp = jnp.pad(bc, (0, 128 - NCLS)).reshape(1, 128)
    w3p = jnp.pad(W3, ((0, 0), (0, 124)))
    b3p = jnp.pad(b3, (0, 124)).reshape(1, 128)
    b1r = b1.reshape(1, ND)
    b2r = b2.reshape(1, ND)

    rr = jnp.concatenate([init_reference[None], inter_references[:LVLS - 1]],
                         axis=0).reshape(LVLS * NBATCH, NQ, 4)

    cls5, coord5 = _run_heads(x, rr, wcp, bcp, W1, b1r, W2, b2r, w3p, b3p,
                              LVLS - 1, LVLS)
    cls04, coord04 = _run_heads(x, rr, wcp, bcp, W1, b1r, W2, b2r, w3p, b3p,
                                0, LVLS - 1, full_out=True)
    outputs_class = lax.dynamic_update_slice(cls04, cls5[None], (5, 0, 0, 0))
    outputs_coord = lax.dynamic_update_slice(coord04, coord5[None],
                                             (5, 0, 0, 0))

    logits5 = cls5.reshape(NBATCH, NFLAT)
    pad = jnp.full((NBATCH, NPAD - NFLAT), -jnp.inf, jnp.float32)
    bits = jnp.concatenate([logits5, pad], axis=1)
    boxes5 = coord5.reshape(NBATCH, NQ * 4)
    img_h = orig_target_sizes[:, 0].astype(jnp.float32)
    img_w = orig_target_sizes[:, 1].astype(jnp.float32)
    wb = jnp.broadcast_to(img_w[:, None], (NBATCH, 16)) * jnp.ones((NBATCH, 16))
    hb = jnp.broadcast_to(img_h[:, None], (NBATCH, 16)) * jnp.ones((NBATCH, 16))
    zz = jnp.zeros((16 * HSTRIDE,), jnp.int32)
    scores_p, labels_p, boxes_p = _run_topk(bits, boxes5, wb, hb, zz)
    boxes_p = boxes_p.reshape(NBATCH, OUTW, 4)
    return (outputs_class, outputs_coord,
            scores_p[:, :KTOP], labels_p[:, :KTOP], boxes_p[:, :KTOP, :])
